# initial kernel scaffold (unmeasured)
import functools

import jax
import jax.numpy as jnp
from jax import lax
from jax.experimental import pallas as pl
from jax.experimental.pallas import tpu as pltpu

N_DEV = 4
B = 2
S = 512
S_PER = 128
D = 512
HD = 256
DH = 64
NH = 4


def _body(x_ref, wq_ref, wk_ref, wv_ref, wo_ref, out_ref,
          xfull, pout, rsbuf,
          ag_send_sems, ag_recv_sems, rs_send_sems, rs_recv_sems):
    me = lax.axis_index("i")

    barrier = pltpu.get_barrier_semaphore()
    for off in (1, 2, 3):
        peer = lax.rem(me + off, N_DEV)
        pl.semaphore_signal(barrier, inc=1, device_id=(peer,),
                            device_id_type=pl.DeviceIdType.MESH)
    pl.semaphore_wait(barrier, N_DEV - 1)

    ag_rdmas = []
    for off in (1, 2, 3):
        peer = lax.rem(me + off, N_DEV)
        rdma = pltpu.make_async_remote_copy(
            src_ref=x_ref,
            dst_ref=xfull.at[me],
            send_sem=ag_send_sems.at[off - 1],
            recv_sem=ag_recv_sems.at[me],
            device_id=(peer,),
            device_id_type=pl.DeviceIdType.MESH,
        )
        rdma.start()
        ag_rdmas.append(rdma)

    for c in range(N_DEV):
        @pl.when(me == c)
        def _(c=c):
            xfull[c] = x_ref[...]

    for off in (1, 2, 3):
        src = lax.rem(me + off, N_DEV)
        pltpu.make_async_remote_copy(
            src_ref=x_ref,
            dst_ref=xfull.at[src],
            send_sem=ag_send_sems.at[0],
            recv_sem=ag_recv_sems.at[src],
            device_id=(src,),
            device_id_type=pl.DeviceIdType.MESH,
        ).wait_recv()

    wq = wq_ref[...]
    wk = wk_ref[...]
    wv = wv_ref[...]
    wo = wo_ref[...]

    pos = lax.broadcasted_iota(jnp.float32, (S, HD), 0)
    lane = lax.broadcasted_iota(jnp.int32, (S, HD), 1)
    d_in = lane % DH
    d_even = (d_in - (d_in % 2)).astype(jnp.float32)
    inv_freq = jnp.exp(d_even * (-jnp.log(10000.0) / DH))
    ang = pos * inv_freq
    cos_t = jnp.cos(ang)
    sin_t = jnp.sin(ang)
    even_mask = (d_in % 2) == 0

    def rot(t):
        t_next = pltpu.roll(t, -1, 1)
        t_prev = pltpu.roll(t, 1, 1)
        t_r = jnp.where(even_mask, -t_next, t_prev)
        return t * cos_t + t_r * sin_t

    for b in range(B):
        xb = jnp.concatenate([xfull[c, b] for c in range(N_DEV)], axis=0)
        q = rot(jnp.dot(xb, wq, preferred_element_type=jnp.float32))
        k = rot(jnp.dot(xb, wk, preferred_element_type=jnp.float32))
        v = jnp.dot(xb, wv, preferred_element_type=jnp.float32)

        ctx_heads = []
        for h in range(NH):
            qh = q[:, h * DH:(h + 1) * DH]
            kh = k[:, h * DH:(h + 1) * DH]
            vh = v[:, h * DH:(h + 1) * DH]
            s = lax.dot_general(qh, kh, (((1,), (1,)), ((), ())),
                                preferred_element_type=jnp.float32) * 0.125
            s = s - jnp.max(s, axis=-1, keepdims=True)
            w = jnp.exp(s)
            w = w / jnp.sum(w, axis=-1, keepdims=True)
            ctx_heads.append(jnp.dot(w, vh, preferred_element_type=jnp.float32))
        ctx = jnp.concatenate(ctx_heads, axis=1)
        ob = jnp.dot(ctx, wo, preferred_element_type=jnp.float32)
        for c in range(N_DEV):
            pout[c, b] = ob[c * S_PER:(c + 1) * S_PER, :]

    for rdma in ag_rdmas:
        rdma.wait_send()

    rs_rdmas = []
    for off in (1, 2, 3):
        peer = lax.rem(me + off, N_DEV)
        rdma = pltpu.make_async_remote_copy(
            src_ref=pout.at[peer],
            dst_ref=rsbuf.at[me],
            send_sem=rs_send_sems.at[off - 1],
            recv_sem=rs_recv_sems.at[me],
            device_id=(peer,),
            device_id_type=pl.DeviceIdType.MESH,
        )
        rdma.start()
        rs_rdmas.append(rdma)

    for c in range(N_DEV):
        @pl.when(me == c)
        def _(c=c):
            rsbuf[c] = pout[c]

    for off in (1, 2, 3):
        src = lax.rem(me + off, N_DEV)
        pltpu.make_async_remote_copy(
            src_ref=pout.at[0],
            dst_ref=rsbuf.at[src],
            send_sem=rs_send_sems.at[0],
            recv_sem=rs_recv_sems.at[src],
            device_id=(src,),
            device_id_type=pl.DeviceIdType.MESH,
        ).wait_recv()

    out_ref[...] = rsbuf[0] + rsbuf[1] + rsbuf[2] + rsbuf[3]

    for rdma in rs_rdmas:
        rdma.wait_send()


def kernel(x, Wq, Wk, Wv, Wo):
    return pl.pallas_call(
        _body,
        out_shape=jax.ShapeDtypeStruct((B, S_PER, D), jnp.float32),
        in_specs=[pl.BlockSpec(memory_space=pltpu.VMEM)] * 5,
        out_specs=pl.BlockSpec(memory_space=pltpu.VMEM),
        scratch_shapes=[
            pltpu.VMEM((N_DEV, B, S_PER, D), jnp.float32),
            pltpu.VMEM((N_DEV, B, S_PER, D), jnp.float32),
            pltpu.VMEM((N_DEV, B, S_PER, D), jnp.float32),
            pltpu.SemaphoreType.DMA((3,)),
            pltpu.SemaphoreType.DMA((N_DEV,)),
            pltpu.SemaphoreType.DMA((3,)),
            pltpu.SemaphoreType.DMA((N_DEV,)),
        ],
        compiler_params=pltpu.CompilerParams(collective_id=0),
    )(x, Wq, Wk, Wv, Wo)


# baseline (device time: 46962 ns/iter reference)
import functools

import jax
import jax.numpy as jnp
from jax import lax
from jax.experimental import pallas as pl
from jax.experimental.pallas import tpu as pltpu

N_DEV = 4
B = 2
S = 512
S_PER = 128
D = 512
HD = 256
DH = 64
NH = 4


def _body(x_ref, wq_ref, wk_ref, wv_ref, wo_ref, out_ref,
          xfull, pout, rsbuf,
          ag_send_sems, ag_recv_sems, rs_send_sems, rs_recv_sems):
    me = lax.axis_index("i")

    barrier = pltpu.get_barrier_semaphore()
    for off in (1, 2, 3):
        peer = lax.rem(me + off, N_DEV)
        pl.semaphore_signal(barrier, inc=1, device_id=(peer,),
                            device_id_type=pl.DeviceIdType.MESH)
    pl.semaphore_wait(barrier, N_DEV - 1)

    ag_rdmas = []
    for off in (1, 2, 3):
        peer = lax.rem(me + off, N_DEV)
        rdma = pltpu.make_async_remote_copy(
            src_ref=x_ref,
            dst_ref=xfull.at[me],
            send_sem=ag_send_sems.at[off - 1],
            recv_sem=ag_recv_sems.at[me],
            device_id=(peer,),
            device_id_type=pl.DeviceIdType.MESH,
        )
        rdma.start()
        ag_rdmas.append(rdma)

    for c in range(N_DEV):
        @pl.when(me == c)
        def _(c=c):
            xfull[c] = x_ref[...]

    for off in (1, 2, 3):
        src = lax.rem(me + off, N_DEV)
        pltpu.make_async_remote_copy(
            src_ref=x_ref,
            dst_ref=xfull.at[src],
            send_sem=ag_send_sems.at[0],
            recv_sem=ag_recv_sems.at[src],
            device_id=(src,),
            device_id_type=pl.DeviceIdType.MESH,
        ).wait_recv()

    wq = wq_ref[...]
    wk = wk_ref[...]
    wv = wv_ref[...]
    wo = wo_ref[...]

    pos = lax.broadcasted_iota(jnp.int32, (S, HD), 0).astype(jnp.float32)
    lane = lax.broadcasted_iota(jnp.int32, (S, HD), 1)
    d_in = lane % DH
    d_even = (d_in - (d_in % 2)).astype(jnp.float32)
    inv_freq = jnp.exp(d_even * (-jnp.log(10000.0) / DH))
    ang = pos * inv_freq
    cos_t = jnp.cos(ang)
    sin_t = jnp.sin(ang)
    even_mask = (d_in % 2) == 0

    def rot(t):
        t_next = pltpu.roll(t, HD - 1, 1)
        t_prev = pltpu.roll(t, 1, 1)
        t_r = jnp.where(even_mask, -t_next, t_prev)
        return t * cos_t + t_r * sin_t

    for b in range(B):
        xb = jnp.concatenate([xfull[c, b] for c in range(N_DEV)], axis=0)
        q = rot(jnp.dot(xb, wq, preferred_element_type=jnp.float32))
        k = rot(jnp.dot(xb, wk, preferred_element_type=jnp.float32))
        v = jnp.dot(xb, wv, preferred_element_type=jnp.float32)

        ctx_heads = []
        for h in range(NH):
            qh = q[:, h * DH:(h + 1) * DH]
            kh = k[:, h * DH:(h + 1) * DH]
            vh = v[:, h * DH:(h + 1) * DH]
            s = lax.dot_general(qh, kh, (((1,), (1,)), ((), ())),
                                preferred_element_type=jnp.float32) * 0.125
            s = s - jnp.max(s, axis=-1, keepdims=True)
            w = jnp.exp(s)
            w = w / jnp.sum(w, axis=-1, keepdims=True)
            ctx_heads.append(jnp.dot(w, vh, preferred_element_type=jnp.float32))
        ctx = jnp.concatenate(ctx_heads, axis=1)
        ob = jnp.dot(ctx, wo, preferred_element_type=jnp.float32)
        for c in range(N_DEV):
            pout[c, b] = ob[c * S_PER:(c + 1) * S_PER, :]

    for rdma in ag_rdmas:
        rdma.wait_send()

    rs_rdmas = []
    for off in (1, 2, 3):
        peer = lax.rem(me + off, N_DEV)
        rdma = pltpu.make_async_remote_copy(
            src_ref=pout.at[peer],
            dst_ref=rsbuf.at[me],
            send_sem=rs_send_sems.at[off - 1],
            recv_sem=rs_recv_sems.at[me],
            device_id=(peer,),
            device_id_type=pl.DeviceIdType.MESH,
        )
        rdma.start()
        rs_rdmas.append(rdma)

    for c in range(N_DEV):
        @pl.when(me == c)
        def _(c=c):
            rsbuf[c] = pout[c]

    for off in (1, 2, 3):
        src = lax.rem(me + off, N_DEV)
        pltpu.make_async_remote_copy(
            src_ref=pout.at[0],
            dst_ref=rsbuf.at[src],
            send_sem=rs_send_sems.at[0],
            recv_sem=rs_recv_sems.at[src],
            device_id=(src,),
            device_id_type=pl.DeviceIdType.MESH,
        ).wait_recv()

    out_ref[...] = rsbuf[0] + rsbuf[1] + rsbuf[2] + rsbuf[3]

    for rdma in rs_rdmas:
        rdma.wait_send()


def kernel(x, Wq, Wk, Wv, Wo):
    return pl.pallas_call(
        _body,
        out_shape=jax.ShapeDtypeStruct((B, S_PER, D), jnp.float32),
        in_specs=[pl.BlockSpec(memory_space=pltpu.VMEM)] * 5,
        out_specs=pl.BlockSpec(memory_space=pltpu.VMEM),
        scratch_shapes=[
            pltpu.VMEM((N_DEV, B, S_PER, D), jnp.float32),
            pltpu.VMEM((N_DEV, B, S_PER, D), jnp.float32),
            pltpu.VMEM((N_DEV, B, S_PER, D), jnp.float32),
            pltpu.SemaphoreType.DMA((3,)),
            pltpu.SemaphoreType.DMA((N_DEV,)),
            pltpu.SemaphoreType.DMA((3,)),
            pltpu.SemaphoreType.DMA((N_DEV,)),
        ],
        compiler_params=pltpu.CompilerParams(collective_id=0),
    )(x, Wq, Wk, Wv, Wo)


# device time: 35401 ns/iter; 1.3266x vs baseline; 1.3266x over previous
import jax
import jax.numpy as jnp
from jax import lax
from jax.experimental import pallas as pl
from jax.experimental.pallas import tpu as pltpu

N_DEV = 4
B = 2
S = 512
S_PER = 128
D = 512
HD = 256
DH = 64
NH = 4


def _body(x_ref, wq_ref, wk_ref, wv_ref, wo_ref, out_ref,
          xfull, pout, rsbuf,
          ag_send_sems, ag_recv_sems, rs_send_sems, rs_recv_sems):
    me = lax.axis_index("i")

    barrier = pltpu.get_barrier_semaphore()
    for off in (1, 2, 3):
        peer = lax.rem(me + off, N_DEV)
        pl.semaphore_signal(barrier, inc=1, device_id=(peer,),
                            device_id_type=pl.DeviceIdType.MESH)
    pl.semaphore_wait(barrier, N_DEV - 1)

    ag_rdmas = []
    for b in range(B):
        for off in (1, 2, 3):
            peer = lax.rem(me + off, N_DEV)
            rdma = pltpu.make_async_remote_copy(
                src_ref=x_ref.at[b],
                dst_ref=xfull.at[b, me],
                send_sem=ag_send_sems.at[b * 3 + off - 1],
                recv_sem=ag_recv_sems.at[b, me],
                device_id=(peer,),
                device_id_type=pl.DeviceIdType.MESH,
            )
            rdma.start()
            ag_rdmas.append(rdma)

    for c in range(N_DEV):
        @pl.when(me == c)
        def _(c=c):
            for b in range(B):
                xfull[b, c] = x_ref[b]

    wq = wq_ref[...]
    wk = wk_ref[...]
    wv = wv_ref[...]
    wo = wo_ref[...]

    pos = lax.broadcasted_iota(jnp.int32, (S, HD), 0).astype(jnp.float32)
    lane = lax.broadcasted_iota(jnp.int32, (S, HD), 1)
    d_in = lane % DH
    d_even = (d_in - (d_in % 2)).astype(jnp.float32)
    inv_freq = jnp.exp(d_even * (-jnp.log(10000.0) / DH))
    ang = pos * inv_freq
    cos_t = jnp.cos(ang)
    sin_t = jnp.sin(ang)
    even_mask = (d_in % 2) == 0

    def rot(t):
        t_next = pltpu.roll(t, HD - 1, 1)
        t_prev = pltpu.roll(t, 1, 1)
        t_r = jnp.where(even_mask, -t_next, t_prev)
        return t * cos_t + t_r * sin_t

    rs_rdmas = []
    for b in range(B):
        for off in (1, 2, 3):
            src = lax.rem(me + off, N_DEV)
            pltpu.make_async_remote_copy(
                src_ref=x_ref.at[b],
                dst_ref=xfull.at[b, src],
                send_sem=ag_send_sems.at[0],
                recv_sem=ag_recv_sems.at[b, src],
                device_id=(src,),
                device_id_type=pl.DeviceIdType.MESH,
            ).wait_recv()

        xb = jnp.concatenate([xfull[b, c] for c in range(N_DEV)], axis=0)
        q = rot(jnp.dot(xb, wq, preferred_element_type=jnp.float32))
        k = rot(jnp.dot(xb, wk, preferred_element_type=jnp.float32))
        v = jnp.dot(xb, wv, preferred_element_type=jnp.float32)

        ctx_heads = []
        for h in range(NH):
            qh = q[:, h * DH:(h + 1) * DH]
            kh = k[:, h * DH:(h + 1) * DH]
            vh = v[:, h * DH:(h + 1) * DH]
            s = lax.dot_general(qh, kh, (((1,), (1,)), ((), ())),
                                preferred_element_type=jnp.float32) * 0.125
            s = s - jnp.max(s, axis=-1, keepdims=True)
            w = jnp.exp(s)
            w = w / jnp.sum(w, axis=-1, keepdims=True)
            ctx_heads.append(jnp.dot(w, vh, preferred_element_type=jnp.float32))
        ctx = jnp.concatenate(ctx_heads, axis=1)
        ob = jnp.dot(ctx, wo, preferred_element_type=jnp.float32)
        for c in range(N_DEV):
            pout[b, c] = ob[c * S_PER:(c + 1) * S_PER, :]

        for off in (1, 2, 3):
            peer = lax.rem(me + off, N_DEV)
            rdma = pltpu.make_async_remote_copy(
                src_ref=pout.at[b, peer],
                dst_ref=rsbuf.at[b, me],
                send_sem=rs_send_sems.at[b * 3 + off - 1],
                recv_sem=rs_recv_sems.at[b, me],
                device_id=(peer,),
                device_id_type=pl.DeviceIdType.MESH,
            )
            rdma.start()
            rs_rdmas.append(rdma)

        for c in range(N_DEV):
            @pl.when(me == c)
            def _(c=c, b=b):
                rsbuf[b, c] = pout[b, c]

    for b in range(B):
        for off in (1, 2, 3):
            src = lax.rem(me + off, N_DEV)
            pltpu.make_async_remote_copy(
                src_ref=pout.at[0, 0],
                dst_ref=rsbuf.at[b, src],
                send_sem=rs_send_sems.at[0],
                recv_sem=rs_recv_sems.at[b, src],
                device_id=(src,),
                device_id_type=pl.DeviceIdType.MESH,
            ).wait_recv()
        out_ref[b] = rsbuf[b, 0] + rsbuf[b, 1] + rsbuf[b, 2] + rsbuf[b, 3]

    for rdma in ag_rdmas:
        rdma.wait_send()
    for rdma in rs_rdmas:
        rdma.wait_send()


def kernel(x, Wq, Wk, Wv, Wo):
    return pl.pallas_call(
        _body,
        out_shape=jax.ShapeDtypeStruct((B, S_PER, D), jnp.float32),
        in_specs=[pl.BlockSpec(memory_space=pltpu.VMEM)] * 5,
        out_specs=pl.BlockSpec(memory_space=pltpu.VMEM),
        scratch_shapes=[
            pltpu.VMEM((B, N_DEV, S_PER, D), jnp.float32),
            pltpu.VMEM((B, N_DEV, S_PER, D), jnp.float32),
            pltpu.VMEM((B, N_DEV, S_PER, D), jnp.float32),
            pltpu.SemaphoreType.DMA((B * 3,)),
            pltpu.SemaphoreType.DMA((B, N_DEV)),
            pltpu.SemaphoreType.DMA((B * 3,)),
            pltpu.SemaphoreType.DMA((B, N_DEV)),
        ],
        compiler_params=pltpu.CompilerParams(collective_id=0),
    )(x, Wq, Wk, Wv, Wo)


# device time: 28276 ns/iter; 1.6608x vs baseline; 1.2520x over previous
import jax
import jax.numpy as jnp
from jax import lax
from jax.experimental import pallas as pl
from jax.experimental.pallas import tpu as pltpu

N_DEV = 4
B = 2
S = 512
S_PER = 128
D = 512
HD = 256
DH = 64
NH = 4

BF = jnp.bfloat16
F32 = jnp.float32


def _body(x_ref, wq_ref, wk_ref, wv_ref, wo_ref, out_ref,
          xsend, xfull, pout, rsbuf,
          ag_send_sems, ag_recv_sems, rs_send_sems, rs_recv_sems):
    me = lax.axis_index("i")

    barrier = pltpu.get_barrier_semaphore()
    for off in (1, 2, 3):
        peer = lax.rem(me + off, N_DEV)
        pl.semaphore_signal(barrier, inc=1, device_id=(peer,),
                            device_id_type=pl.DeviceIdType.MESH)
    pl.semaphore_wait(barrier, N_DEV - 1)

    for b in range(B):
        xsend[b] = x_ref[b].astype(BF)

    ag_rdmas = []
    for b in range(B):
        for off in (1, 2, 3):
            peer = lax.rem(me + off, N_DEV)
            rdma = pltpu.make_async_remote_copy(
                src_ref=xsend.at[b],
                dst_ref=xfull.at[b, me],
                send_sem=ag_send_sems.at[b * 3 + off - 1],
                recv_sem=ag_recv_sems.at[b, me],
                device_id=(peer,),
                device_id_type=pl.DeviceIdType.MESH,
            )
            rdma.start()
            ag_rdmas.append(rdma)

    for c in range(N_DEV):
        @pl.when(me == c)
        def _(c=c):
            for b in range(B):
                xfull[b, c] = xsend[b]

    wq = wq_ref[...].astype(BF)
    wk = wk_ref[...].astype(BF)
    wv = wv_ref[...].astype(BF)
    wo = wo_ref[...].astype(BF)

    pos = lax.broadcasted_iota(jnp.int32, (S, HD), 0).astype(F32)
    lane = lax.broadcasted_iota(jnp.int32, (S, HD), 1)
    d_in = lane % DH
    d_even = (d_in - (d_in % 2)).astype(F32)
    inv_freq = jnp.exp(d_even * (-jnp.log(10000.0) / DH))
    ang = pos * inv_freq
    cos_t = jnp.cos(ang)
    sin_t = jnp.sin(ang)
    even_mask = (d_in % 2) == 0

    def rot(t):
        t_next = pltpu.roll(t, HD - 1, 1)
        t_prev = pltpu.roll(t, 1, 1)
        t_r = jnp.where(even_mask, -t_next, t_prev)
        return t * cos_t + t_r * sin_t

    rs_rdmas = []
    for b in range(B):
        for off in (1, 2, 3):
            src = lax.rem(me + off, N_DEV)
            pltpu.make_async_remote_copy(
                src_ref=xsend.at[b],
                dst_ref=xfull.at[b, src],
                send_sem=ag_send_sems.at[0],
                recv_sem=ag_recv_sems.at[b, src],
                device_id=(src,),
                device_id_type=pl.DeviceIdType.MESH,
            ).wait_recv()

        xb = jnp.concatenate([xfull[b, c] for c in range(N_DEV)], axis=0)
        q = rot(jnp.dot(xb, wq, preferred_element_type=F32)).astype(BF)
        k = rot(jnp.dot(xb, wk, preferred_element_type=F32)).astype(BF)
        v = jnp.dot(xb, wv, preferred_element_type=F32).astype(BF)

        ctx_heads = []
        for h in range(NH):
            qh = q[:, h * DH:(h + 1) * DH]
            kh = k[:, h * DH:(h + 1) * DH]
            vh = v[:, h * DH:(h + 1) * DH]
            s = lax.dot_general(qh, kh, (((1,), (1,)), ((), ())),
                                preferred_element_type=F32) * 0.125
            s = s - jnp.max(s, axis=-1, keepdims=True)
            w = jnp.exp(s)
            w = (w / jnp.sum(w, axis=-1, keepdims=True)).astype(BF)
            ctx_heads.append(
                jnp.dot(w, vh, preferred_element_type=F32).astype(BF))
        ctx = jnp.concatenate(ctx_heads, axis=1)
        ob = jnp.dot(ctx, wo, preferred_element_type=F32)
        for c in range(N_DEV):
            pout[b, c] = ob[c * S_PER:(c + 1) * S_PER, :].astype(BF)

        for off in (1, 2, 3):
            peer = lax.rem(me + off, N_DEV)
            rdma = pltpu.make_async_remote_copy(
                src_ref=pout.at[b, peer],
                dst_ref=rsbuf.at[b, me],
                send_sem=rs_send_sems.at[b * 3 + off - 1],
                recv_sem=rs_recv_sems.at[b, me],
                device_id=(peer,),
                device_id_type=pl.DeviceIdType.MESH,
            )
            rdma.start()
            rs_rdmas.append(rdma)

        for c in range(N_DEV):
            @pl.when(me == c)
            def _(c=c, b=b):
                rsbuf[b, c] = pout[b, c]

    for b in range(B):
        for off in (1, 2, 3):
            src = lax.rem(me + off, N_DEV)
            pltpu.make_async_remote_copy(
                src_ref=pout.at[0, 0],
                dst_ref=rsbuf.at[b, src],
                send_sem=rs_send_sems.at[0],
                recv_sem=rs_recv_sems.at[b, src],
                device_id=(src,),
                device_id_type=pl.DeviceIdType.MESH,
            ).wait_recv()
        out_ref[b] = (rsbuf[b, 0].astype(F32) + rsbuf[b, 1].astype(F32)
                      + rsbuf[b, 2].astype(F32) + rsbuf[b, 3].astype(F32))

    for rdma in ag_rdmas:
        rdma.wait_send()
    for rdma in rs_rdmas:
        rdma.wait_send()


def kernel(x, Wq, Wk, Wv, Wo):
    return pl.pallas_call(
        _body,
        out_shape=jax.ShapeDtypeStruct((B, S_PER, D), jnp.float32),
        in_specs=[pl.BlockSpec(memory_space=pltpu.VMEM)] * 5,
        out_specs=pl.BlockSpec(memory_space=pltpu.VMEM),
        scratch_shapes=[
            pltpu.VMEM((B, S_PER, D), BF),
            pltpu.VMEM((B, N_DEV, S_PER, D), BF),
            pltpu.VMEM((B, N_DEV, S_PER, D), BF),
            pltpu.VMEM((B, N_DEV, S_PER, D), BF),
            pltpu.SemaphoreType.DMA((B * 3,)),
            pltpu.SemaphoreType.DMA((B, N_DEV)),
            pltpu.SemaphoreType.DMA((B * 3,)),
            pltpu.SemaphoreType.DMA((B, N_DEV)),
        ],
        compiler_params=pltpu.CompilerParams(collective_id=0),
    )(x, Wq, Wk, Wv, Wo)
